# Initial kernel scaffold; baseline (speedup 1.0000x reference)
import functools

import jax
import jax.numpy as jnp
from jax import lax
from jax.experimental import pallas as pl
from jax.experimental.pallas import tpu as pltpu
from jax.experimental.pallas import tpu_sc as plsc

N_NODES = 10000
N_EDGES = 320000
HIDDEN = 128

_NUM_WORKERS = 32
_EDGES_PER_WORKER = N_EDGES // _NUM_WORKERS
_LANES = 16


def _scores_body(x_ref, w_ref, s_ref):
    x = x_ref[...]
    w = w_ref[...]
    s0 = jnp.sum(x * w[0:1, :], axis=1, keepdims=True)
    s1 = jnp.sum(x * w[1:2, :], axis=1, keepdims=True)
    s_ref[...] = jnp.concatenate([s0, s1], axis=1)


def _node_scores(x, w):
    return pl.pallas_call(
        _scores_body,
        out_shape=jax.ShapeDtypeStruct((N_NODES, 2), jnp.float32),
    )(x, w)


def _make_edge_kernel():
    mesh = plsc.VectorSubcoreMesh(core_axis_name="c", subcore_axis_name="s")

    @functools.partial(
        pl.kernel,
        mesh=mesh,
        out_type=jax.ShapeDtypeStruct((N_EDGES,), jnp.float32),
        scratch_types=[
            pltpu.VMEM((_EDGES_PER_WORKER,), jnp.int32),
            pltpu.VMEM((_EDGES_PER_WORKER,), jnp.int32),
            pltpu.VMEM((N_NODES, 2), jnp.float32),
            pltpu.VMEM((_EDGES_PER_WORKER,), jnp.float32),
        ],
    )
    def edge_kernel(row_hbm, col_hbm, s_hbm, out_hbm, row_v, col_v, s_v, out_v):
        wid = lax.axis_index("s") * 2 + lax.axis_index("c")
        base = wid * _EDGES_PER_WORKER
        pltpu.sync_copy(row_hbm.at[pl.ds(base, _EDGES_PER_WORKER)], row_v)
        pltpu.sync_copy(col_hbm.at[pl.ds(base, _EDGES_PER_WORKER)], col_v)
        pltpu.sync_copy(s_hbm, s_v)

        zero = jnp.zeros((_LANES,), jnp.int32)
        one = zero + 1

        def body(i, carry):
            off = i * _LANES
            r = row_v[pl.ds(off, _LANES)]
            c = col_v[pl.ds(off, _LANES)]
            a = plsc.load_gather(s_v, [r, zero])
            b = plsc.load_gather(s_v, [c, one])
            z = a + b
            out_v[pl.ds(off, _LANES)] = 1.0 / (1.0 + jnp.exp(-z))
            return carry

        lax.fori_loop(0, _EDGES_PER_WORKER // _LANES, body, 0)
        pltpu.sync_copy(out_v, out_hbm.at[pl.ds(base, _EDGES_PER_WORKER)])

    return edge_kernel


_edge_kernel = _make_edge_kernel()


def kernel(edge_index, x, att_weight):
    ei = edge_index.astype(jnp.int32)
    row = ei[0]
    col = ei[1]
    w = att_weight.reshape(2, HIDDEN)
    s = _node_scores(x, w)
    out = _edge_kernel(row, col, s)
    return out.reshape(N_EDGES, 1)


# speedup vs baseline: 24.6240x; 24.6240x over previous
import functools

import jax
import jax.numpy as jnp
from jax import lax
from jax.experimental import pallas as pl
from jax.experimental.pallas import tpu as pltpu
from jax.experimental.pallas import tpu_sc as plsc

N_NODES = 10000
N_EDGES = 320000
HIDDEN = 128

_NUM_WORKERS = 32
_EDGES_PER_WORKER = N_EDGES // _NUM_WORKERS
_LANES = 16


def _scores_body(x_ref, w_ref, s_ref):
    x = x_ref[...]
    w = w_ref[...]
    s0 = jnp.sum(x * w[0:1, :], axis=1, keepdims=True)
    s1 = jnp.sum(x * w[1:2, :], axis=1, keepdims=True)
    s_ref[...] = jnp.concatenate([s0, s1], axis=1)


def _node_scores(x, w):
    return pl.pallas_call(
        _scores_body,
        out_shape=jax.ShapeDtypeStruct((N_NODES, 2), jnp.float32),
    )(x, w)


def _make_edge_kernel():
    mesh = plsc.VectorSubcoreMesh(core_axis_name="c", subcore_axis_name="s")

    @functools.partial(
        pl.kernel,
        mesh=mesh,
        out_type=jax.ShapeDtypeStruct((N_EDGES,), jnp.float32),
        compiler_params=pltpu.CompilerParams(needs_layout_passes=False),
        scratch_types=[
            pltpu.VMEM((_EDGES_PER_WORKER,), jnp.int32),
            pltpu.VMEM((_EDGES_PER_WORKER,), jnp.int32),
            pltpu.VMEM((2 * N_NODES,), jnp.float32),
            pltpu.VMEM((_EDGES_PER_WORKER,), jnp.float32),
        ],
    )
    def edge_kernel(row_hbm, col_hbm, s_hbm, out_hbm, row_v, col_v, s_v, out_v):
        wid = lax.axis_index("s") * 2 + lax.axis_index("c")
        base = wid * _EDGES_PER_WORKER
        pltpu.sync_copy(row_hbm.at[pl.ds(base, _EDGES_PER_WORKER)], row_v)
        pltpu.sync_copy(col_hbm.at[pl.ds(base, _EDGES_PER_WORKER)], col_v)
        pltpu.sync_copy(s_hbm, s_v)

        def body(i, carry):
            off = i * _LANES
            r = row_v[pl.ds(off, _LANES)]
            c = col_v[pl.ds(off, _LANES)]
            a = plsc.load_gather(s_v, [r * 2])
            b = plsc.load_gather(s_v, [c * 2 + 1])
            z = a + b
            out_v[pl.ds(off, _LANES)] = 1.0 / (1.0 + jnp.exp(-z))
            return carry

        lax.fori_loop(0, _EDGES_PER_WORKER // _LANES, body, 0)
        pltpu.sync_copy(out_v, out_hbm.at[pl.ds(base, _EDGES_PER_WORKER)])

    return edge_kernel


_edge_kernel = _make_edge_kernel()


def kernel(edge_index, x, att_weight):
    ei = edge_index.astype(jnp.int32)
    row = ei[0]
    col = ei[1]
    w = att_weight.reshape(2, HIDDEN)
    s = _node_scores(x, w).reshape(2 * N_NODES)
    out = _edge_kernel(row, col, s)
    return out.reshape(N_EDGES, 1)


# speedup vs baseline: 29.8387x; 1.2118x over previous
import functools

import jax
import jax.numpy as jnp
from jax import lax
from jax.experimental import pallas as pl
from jax.experimental.pallas import tpu as pltpu
from jax.experimental.pallas import tpu_sc as plsc

N_NODES = 10000
N_EDGES = 320000
HIDDEN = 128

_NUM_WORKERS = 32
_EDGES_PER_WORKER = N_EDGES // _NUM_WORKERS
_LANES = 16


def _scores_body(x_ref, w_ref, s_ref):
    x = x_ref[...]
    w = w_ref[...]
    s0 = jnp.sum(x * w[0:1, :], axis=1, keepdims=True)
    s1 = jnp.sum(x * w[1:2, :], axis=1, keepdims=True)
    s_ref[...] = jnp.concatenate([s0, s1], axis=1)


def _node_scores(x, w):
    return pl.pallas_call(
        _scores_body,
        out_shape=jax.ShapeDtypeStruct((N_NODES, 2), jnp.float32),
    )(x, w)


def _make_edge_kernel():
    mesh = plsc.VectorSubcoreMesh(core_axis_name="c", subcore_axis_name="s")

    @functools.partial(
        pl.kernel,
        mesh=mesh,
        out_type=jax.ShapeDtypeStruct((N_EDGES,), jnp.float32),
        compiler_params=pltpu.CompilerParams(needs_layout_passes=False),
        scratch_types=[
            pltpu.VMEM((_EDGES_PER_WORKER,), jnp.int32),
            pltpu.VMEM((_EDGES_PER_WORKER,), jnp.int32),
            pltpu.VMEM((2 * N_NODES,), jnp.float32),
            pltpu.VMEM((_EDGES_PER_WORKER,), jnp.float32),
            pltpu.SemaphoreType.DMA,
            pltpu.SemaphoreType.DMA,
            pltpu.SemaphoreType.DMA,
        ],
    )
    def edge_kernel(row_hbm, col_hbm, s_hbm, out_hbm, row_v, col_v, s_v, out_v,
                    sem0, sem1, sem2):
        wid = lax.axis_index("s") * 2 + lax.axis_index("c")
        base = wid * _EDGES_PER_WORKER
        cp0 = pltpu.async_copy(row_hbm.at[pl.ds(base, _EDGES_PER_WORKER)], row_v, sem0)
        cp1 = pltpu.async_copy(col_hbm.at[pl.ds(base, _EDGES_PER_WORKER)], col_v, sem1)
        cp2 = pltpu.async_copy(s_hbm, s_v, sem2)
        cp0.wait()
        cp1.wait()
        cp2.wait()

        @plsc.parallel_loop(0, _EDGES_PER_WORKER, step=_LANES, unroll=8)
        def body(off):
            r = row_v[pl.ds(off, _LANES)]
            c = col_v[pl.ds(off, _LANES)]
            a = plsc.load_gather(s_v, [r * 2])
            b = plsc.load_gather(s_v, [c * 2 + 1])
            z = a + b
            out_v[pl.ds(off, _LANES)] = 1.0 / (1.0 + jnp.exp(-z))

        pltpu.sync_copy(out_v, out_hbm.at[pl.ds(base, _EDGES_PER_WORKER)])

    return edge_kernel


_edge_kernel = _make_edge_kernel()


def kernel(edge_index, x, att_weight):
    ei = edge_index.astype(jnp.int32)
    row = ei[0]
    col = ei[1]
    w = att_weight.reshape(2, HIDDEN)
    s = _node_scores(x, w).reshape(2 * N_NODES)
    out = _edge_kernel(row, col, s)
    return out.reshape(N_EDGES, 1)


# speedup vs baseline: 37.2787x; 1.2493x over previous
import functools

import jax
import jax.numpy as jnp
from jax import lax
from jax.experimental import pallas as pl
from jax.experimental.pallas import tpu as pltpu
from jax.experimental.pallas import tpu_sc as plsc

N_NODES = 10000
N_EDGES = 320000
HIDDEN = 128

_NUM_WORKERS = 32
_EDGES_PER_WORKER = N_EDGES // _NUM_WORKERS
_LANES = 16


def _scores_body(x_ref, w_ref, s_ref):
    x = x_ref[...]
    w = w_ref[...]
    s0 = jnp.sum(x * w[0:1, :], axis=1, keepdims=True)
    s1 = jnp.sum(x * w[1:2, :], axis=1, keepdims=True)
    s_ref[...] = jnp.concatenate([s0, s1], axis=1)


def _node_scores(x, w):
    return pl.pallas_call(
        _scores_body,
        out_shape=jax.ShapeDtypeStruct((N_NODES, 2), jnp.float32),
    )(x, w)


def _make_edge_kernel():
    mesh = plsc.VectorSubcoreMesh(core_axis_name="c", subcore_axis_name="s")

    @functools.partial(
        pl.kernel,
        mesh=mesh,
        out_type=jax.ShapeDtypeStruct((N_EDGES,), jnp.float32),
        compiler_params=pltpu.CompilerParams(needs_layout_passes=False),
        scratch_types=[
            pltpu.VMEM((_EDGES_PER_WORKER,), jnp.int32),
            pltpu.VMEM((_EDGES_PER_WORKER,), jnp.int32),
            pltpu.VMEM((2 * N_NODES,), jnp.float32),
            pltpu.VMEM((_EDGES_PER_WORKER,), jnp.float32),
            pltpu.SemaphoreType.DMA,
            pltpu.SemaphoreType.DMA,
            pltpu.SemaphoreType.DMA,
        ],
    )
    def edge_kernel(ei_hbm, s_hbm, out_hbm, row_v, col_v, s_v, out_v,
                    sem0, sem1, sem2):
        wid = lax.axis_index("s") * 2 + lax.axis_index("c")
        base = wid * _EDGES_PER_WORKER
        cp0 = pltpu.async_copy(ei_hbm.at[pl.ds(base, _EDGES_PER_WORKER)], row_v, sem0)
        cp1 = pltpu.async_copy(ei_hbm.at[pl.ds(N_EDGES + base, _EDGES_PER_WORKER)], col_v, sem1)
        cp2 = pltpu.async_copy(s_hbm, s_v, sem2)
        cp0.wait()
        cp1.wait()
        cp2.wait()

        @plsc.parallel_loop(0, _EDGES_PER_WORKER, step=_LANES, unroll=8)
        def body(off):
            r = row_v[pl.ds(off, _LANES)]
            c = col_v[pl.ds(off, _LANES)]
            a = plsc.load_gather(s_v, [r * 2])
            b = plsc.load_gather(s_v, [c * 2 + 1])
            z = a + b
            out_v[pl.ds(off, _LANES)] = 1.0 / (1.0 + jnp.exp(-z))

        pltpu.sync_copy(out_v, out_hbm.at[pl.ds(base, _EDGES_PER_WORKER)])

    return edge_kernel


_edge_kernel = _make_edge_kernel()


def kernel(edge_index, x, att_weight):
    ei_flat = edge_index.astype(jnp.int32).reshape(2 * N_EDGES)
    w = att_weight.reshape(2, HIDDEN)
    s = _node_scores(x, w).reshape(2 * N_NODES)
    return _edge_kernel(ei_flat, s).reshape(N_EDGES, 1)
